# baseline (speedup 1.0000x reference)
import jax
import jax.numpy as jnp
from jax import lax
from jax.experimental import pallas as pl
from jax.experimental.pallas import tpu as pltpu
from jax.experimental.pallas import tpu_sc as plsc

_NC = 2
_NS = 16
_NW = _NC * _NS
_CHUNK = 128
_D = 64


def _emb_body(idx_hbm, table_hbm, out_hbm, idx_v, rows_v, sem):
    wid = lax.axis_index("s") * _NC + lax.axis_index("c")
    n_chunks = idx_v.shape[0]
    base = wid * (n_chunks * _CHUNK)
    pltpu.sync_copy(idx_hbm.at[wid], idx_v)

    def step(c, carry):
        pltpu.async_copy(table_hbm.at[idx_v.at[c]], rows_v, sem).wait()
        pltpu.sync_copy(rows_v, out_hbm.at[pl.ds(base + c * _CHUNK, _CHUNK)])
        return carry

    lax.fori_loop(0, n_chunks, step, 0)


def kernel(tokens, weight):
    s0, s1 = tokens.shape
    b = s0 * s1
    n_chunks = b // (_NW * _CHUNK)
    idx = tokens.reshape(_NW, n_chunks, _CHUNK).astype(jnp.int32)
    mesh = plsc.VectorSubcoreMesh(core_axis_name="c", subcore_axis_name="s")
    out = pl.kernel(
        _emb_body,
        out_type=jax.ShapeDtypeStruct((b, _D), jnp.float32),
        mesh=mesh,
        compiler_params=pltpu.CompilerParams(use_tc_tiling_on_sc=False),
        scratch_types=[
            pltpu.VMEM((n_chunks, _CHUNK), jnp.int32),
            pltpu.VMEM((_CHUNK, _D), jnp.float32),
            pltpu.SemaphoreType.DMA,
        ],
    )(idx, weight)
    return out.reshape(s0, s1, _D)


# speedup vs baseline: 1.1135x; 1.1135x over previous
import jax
import jax.numpy as jnp
from jax import lax
from jax.experimental import pallas as pl
from jax.experimental.pallas import tpu as pltpu
from jax.experimental.pallas import tpu_sc as plsc

_NC = 2
_NS = 16
_NW = _NC * _NS
_CHUNK = 128
_G = 4
_D = 64


def _emb_body(idx_hbm, table_hbm, out_hbm, idx_v, rows0, rows1, gsem, wsem0, wsem1):
    wid = lax.axis_index("s") * _NC + lax.axis_index("c")
    n_chunks = idx_v.shape[0]
    n_groups = n_chunks // _G
    grows = _G * _CHUNK
    base = wid * (n_chunks * _CHUNK)
    pltpu.sync_copy(idx_hbm.at[wid], idx_v)

    def pair(p, carry):
        for b, (rows, wsem) in enumerate(((rows0, wsem0), (rows1, wsem1))):
            g = 2 * p + b
            @pl.when(g >= 2)
            def _():
                pltpu.make_async_copy(
                    rows, out_hbm.at[pl.ds(base, grows)], wsem).wait()

            for j in range(_G):
                pltpu.async_copy(
                    table_hbm.at[idx_v.at[g * _G + j]],
                    rows.at[pl.ds(j * _CHUNK, _CHUNK)], gsem)
            for j in range(_G):
                pltpu.make_async_copy(
                    table_hbm.at[idx_v.at[0]],
                    rows.at[pl.ds(0, _CHUNK)], gsem).wait()
            pltpu.async_copy(
                rows, out_hbm.at[pl.ds(base + g * grows, grows)], wsem)
        return carry

    lax.fori_loop(0, n_groups // 2, pair, 0)
    pltpu.make_async_copy(rows0, out_hbm.at[pl.ds(base, grows)], wsem0).wait()
    pltpu.make_async_copy(rows1, out_hbm.at[pl.ds(base, grows)], wsem1).wait()


def kernel(tokens, weight):
    s0, s1 = tokens.shape
    b = s0 * s1
    n_chunks = b // (_NW * _CHUNK)
    idx = tokens.reshape(_NW, n_chunks, _CHUNK).astype(jnp.int32)
    mesh = plsc.VectorSubcoreMesh(core_axis_name="c", subcore_axis_name="s")
    out = pl.kernel(
        _emb_body,
        out_type=jax.ShapeDtypeStruct((b, _D), jnp.float32),
        mesh=mesh,
        compiler_params=pltpu.CompilerParams(use_tc_tiling_on_sc=False),
        scratch_types=[
            pltpu.VMEM((n_chunks, _CHUNK), jnp.int32),
            pltpu.VMEM((_G * _CHUNK, _D), jnp.float32),
            pltpu.VMEM((_G * _CHUNK, _D), jnp.float32),
            pltpu.SemaphoreType.DMA,
            pltpu.SemaphoreType.DMA,
            pltpu.SemaphoreType.DMA,
        ],
    )(idx, weight)
    return out.reshape(s0, s1, _D)


# speedup vs baseline: 1.1144x; 1.0008x over previous
import jax
import jax.numpy as jnp
from jax import lax
from jax.experimental import pallas as pl
from jax.experimental.pallas import tpu as pltpu
from jax.experimental.pallas import tpu_sc as plsc

_NC = 2
_NS = 16
_NW = _NC * _NS
_CHUNK = 128
_G = 4
_D = 64


def _emb_body(idx_hbm, table_hbm, out_hbm, idx_v, rows0, rows1, gs0, gs1, ws0, ws1):
    wid = lax.axis_index("s") * _NC + lax.axis_index("c")
    n_chunks = idx_v.shape[0]
    n_groups = n_chunks // _G
    grows = _G * _CHUNK
    base = wid * (n_chunks * _CHUNK)
    pltpu.sync_copy(idx_hbm.at[wid], idx_v)

    def fire(g, rows, gsem):
        for j in range(_G):
            pltpu.async_copy(
                table_hbm.at[idx_v.at[g * _G + j]],
                rows.at[pl.ds(j * _CHUNK, _CHUNK)], gsem)

    def drain_gathers(rows, gsem):
        for j in range(_G):
            pltpu.make_async_copy(
                table_hbm.at[idx_v.at[0]],
                rows.at[pl.ds(0, _CHUNK)], gsem).wait()

    def write(g, rows, wsem):
        pltpu.async_copy(
            rows, out_hbm.at[pl.ds(base + g * grows, grows)], wsem)

    def wait_write(rows, wsem):
        pltpu.make_async_copy(rows, out_hbm.at[pl.ds(base, grows)], wsem).wait()

    fire(0, rows0, gs0)
    last = n_groups - 1

    def pair(p, carry):
        r_odd = 2 * p + 1

        @pl.when(p >= 1)
        def _():
            wait_write(rows1, ws1)
        fire(r_odd, rows1, gs1)
        drain_gathers(rows0, gs0)
        write(r_odd - 1, rows0, ws0)

        wait_write(rows0, ws0)
        fire(r_odd + 1, rows0, gs0)
        drain_gathers(rows1, gs1)
        write(r_odd, rows1, ws1)
        return carry

    lax.fori_loop(0, (n_groups - 2) // 2, pair, 0)
    wait_write(rows1, ws1)
    fire(last, rows1, gs1)
    drain_gathers(rows0, gs0)
    write(last - 1, rows0, ws0)
    drain_gathers(rows1, gs1)
    write(last, rows1, ws1)
    wait_write(rows0, ws0)
    wait_write(rows1, ws1)


def kernel(tokens, weight):
    s0, s1 = tokens.shape
    b = s0 * s1
    n_chunks = b // (_NW * _CHUNK)
    idx = tokens.reshape(_NW, n_chunks, _CHUNK).astype(jnp.int32)
    mesh = plsc.VectorSubcoreMesh(core_axis_name="c", subcore_axis_name="s")
    out = pl.kernel(
        _emb_body,
        out_type=jax.ShapeDtypeStruct((b, _D), jnp.float32),
        mesh=mesh,
        compiler_params=pltpu.CompilerParams(use_tc_tiling_on_sc=False),
        scratch_types=[
            pltpu.VMEM((n_chunks, _CHUNK), jnp.int32),
            pltpu.VMEM((_G * _CHUNK, _D), jnp.float32),
            pltpu.VMEM((_G * _CHUNK, _D), jnp.float32),
            pltpu.SemaphoreType.DMA,
            pltpu.SemaphoreType.DMA,
            pltpu.SemaphoreType.DMA,
            pltpu.SemaphoreType.DMA,
        ],
    )(idx, weight)
    return out.reshape(s0, s1, _D)


# speedup vs baseline: 1.1165x; 1.0019x over previous
import jax
import jax.numpy as jnp
from jax import lax
from jax.experimental import pallas as pl
from jax.experimental.pallas import tpu as pltpu
from jax.experimental.pallas import tpu_sc as plsc

_NC = 2
_NS = 16
_NW = _NC * _NS
_CHUNK = 512
_G = 1
_D = 64


def _emb_body(idx_hbm, table_hbm, out_hbm, idx_v, rows0, rows1, gs0, gs1, ws0, ws1):
    wid = lax.axis_index("s") * _NC + lax.axis_index("c")
    n_chunks = idx_v.shape[0]
    n_groups = n_chunks // _G
    grows = _G * _CHUNK
    base = wid * (n_chunks * _CHUNK)
    pltpu.sync_copy(idx_hbm.at[wid], idx_v)

    def fire(g, rows, gsem):
        for j in range(_G):
            pltpu.async_copy(
                table_hbm.at[idx_v.at[g * _G + j]],
                rows.at[pl.ds(j * _CHUNK, _CHUNK)], gsem)

    def drain_gathers(rows, gsem):
        for j in range(_G):
            pltpu.make_async_copy(
                table_hbm.at[idx_v.at[0]],
                rows.at[pl.ds(0, _CHUNK)], gsem).wait()

    def write(g, rows, wsem):
        pltpu.async_copy(
            rows, out_hbm.at[pl.ds(base + g * grows, grows)], wsem)

    def wait_write(rows, wsem):
        pltpu.make_async_copy(rows, out_hbm.at[pl.ds(base, grows)], wsem).wait()

    fire(0, rows0, gs0)
    last = n_groups - 1

    def pair(p, carry):
        r_odd = 2 * p + 1

        @pl.when(p >= 1)
        def _():
            wait_write(rows1, ws1)
        fire(r_odd, rows1, gs1)
        drain_gathers(rows0, gs0)
        write(r_odd - 1, rows0, ws0)

        wait_write(rows0, ws0)
        fire(r_odd + 1, rows0, gs0)
        drain_gathers(rows1, gs1)
        write(r_odd, rows1, ws1)
        return carry

    lax.fori_loop(0, (n_groups - 2) // 2, pair, 0)
    wait_write(rows1, ws1)
    fire(last, rows1, gs1)
    drain_gathers(rows0, gs0)
    write(last - 1, rows0, ws0)
    drain_gathers(rows1, gs1)
    write(last, rows1, ws1)
    wait_write(rows0, ws0)
    wait_write(rows1, ws1)


def kernel(tokens, weight):
    s0, s1 = tokens.shape
    b = s0 * s1
    n_chunks = b // (_NW * _CHUNK)
    idx = tokens.reshape(_NW, n_chunks, _CHUNK).astype(jnp.int32)
    mesh = plsc.VectorSubcoreMesh(core_axis_name="c", subcore_axis_name="s")
    out = pl.kernel(
        _emb_body,
        out_type=jax.ShapeDtypeStruct((b, _D), jnp.float32),
        mesh=mesh,
        compiler_params=pltpu.CompilerParams(use_tc_tiling_on_sc=False),
        scratch_types=[
            pltpu.VMEM((n_chunks, _CHUNK), jnp.int32),
            pltpu.VMEM((_G * _CHUNK, _D), jnp.float32),
            pltpu.VMEM((_G * _CHUNK, _D), jnp.float32),
            pltpu.SemaphoreType.DMA,
            pltpu.SemaphoreType.DMA,
            pltpu.SemaphoreType.DMA,
            pltpu.SemaphoreType.DMA,
        ],
    )(idx, weight)
    return out.reshape(s0, s1, _D)
